# baseline (speedup 1.0000x reference)
import functools

import jax
import jax.numpy as jnp
from jax import lax
from jax.experimental import pallas as pl
from jax.experimental.pallas import tpu as pltpu
from jax.experimental.pallas import tpu_sc as plsc


def _sc_dots(v_i, v_j, neg_flat, second, context, NW, BW, K, L):
    B = v_i.shape[0]
    D = second.shape[1]
    NC = NW // 16
    NCH = D // L
    NI = 2 + K
    ND = 1 + K

    mesh = plsc.VectorSubcoreMesh(core_axis_name="c", subcore_axis_name="s",
                                  num_cores=NC)

    @functools.partial(
        pl.kernel,
        mesh=mesh,
        out_type=jax.ShapeDtypeStruct((NW * ND, BW, L), jnp.float32),
        compiler_params=pltpu.CompilerParams(use_tc_tiling_on_sc=False),
        scratch_types=[
            pltpu.VMEM((NI, BW), jnp.int32),
            pltpu.VMEM((BW, D), jnp.float32),
            pltpu.VMEM((BW, D), jnp.float32),
            pltpu.VMEM((K, BW, D), jnp.float32),
            pltpu.VMEM((ND, BW, L), jnp.float32),
            pltpu.SemaphoreType.DMA,
            pltpu.SemaphoreType.DMA,
            pltpu.SemaphoreType.DMA,
        ],
    )
    def k(vi_hbm, vj_hbm, neg_hbm, second_hbm, context_hbm, out_hbm,
          idx_v, vi_rows, vj_rows, neg_rows, out_v, isem, sem, wsem):
        wid = lax.axis_index("s") * NC + lax.axis_index("c")
        base = wid * BW

        scps = [
            pltpu.async_copy(vi_hbm.at[pl.ds(base, BW)], idx_v.at[0], isem),
            pltpu.async_copy(vj_hbm.at[pl.ds(base, BW)], idx_v.at[1], isem),
        ]
        for kk in range(K):
            scps.append(
                pltpu.async_copy(neg_hbm.at[pl.ds(kk * B + base, BW)],
                                 idx_v.at[2 + kk], isem))
        for cp in scps:
            cp.wait()

        cps = [
            pltpu.async_copy(second_hbm.at[idx_v.at[0]], vi_rows, sem),
            pltpu.async_copy(context_hbm.at[idx_v.at[1]], vj_rows, sem),
        ]
        for kk in range(K):
            cps.append(
                pltpu.async_copy(context_hbm.at[idx_v.at[2 + kk]],
                                 neg_rows.at[kk], sem))
        cps[0].wait()
        cps[1].wait()

        @plsc.parallel_loop(0, BW, unroll=16)
        def pos_body(g):
            acc = vi_rows[g, pl.ds(0, L)] * vj_rows[g, pl.ds(0, L)]
            for c in range(1, NCH):
                acc = acc + (vi_rows[g, pl.ds(c * L, L)]
                             * vj_rows[g, pl.ds(c * L, L)])
            out_v[0, g, :] = acc

        for kk in range(K):
            cps[2 + kk].wait()

            @plsc.parallel_loop(0, BW, unroll=16)
            def neg_body(g, _kk=kk):
                acc = (vi_rows[g, pl.ds(0, L)]
                       * neg_rows[_kk, g, pl.ds(0, L)])
                for c in range(1, NCH):
                    acc = acc + (vi_rows[g, pl.ds(c * L, L)]
                                 * neg_rows[_kk, g, pl.ds(c * L, L)])
                out_v[1 + _kk, g, :] = acc

        pltpu.sync_copy(out_v, out_hbm.at[pl.ds(wid * ND, ND)])

    return k(v_i, v_j, neg_flat, second, context)


def _tc_finalize(x, batch, num_dots, block_rows):
    R, C = x.shape
    L = 16
    G = C // L

    def body(x_ref, o_ref):
        xs = x_ref[...]
        col = lax.broadcasted_iota(jnp.int32, (C, G), 0)
        grp = lax.broadcasted_iota(jnp.int32, (C, G), 1)
        a = (col // L == grp).astype(jnp.float32)
        s = jnp.dot(xs, a, preferred_element_type=jnp.float32)

        row = lax.broadcasted_iota(jnp.int32, (R, G), 0)
        d = (row // block_rows) % num_dots
        v = jnp.where(d == 0, s, -s)
        acc = jnp.minimum(v, 0.0) - jnp.log1p(jnp.exp(-jnp.abs(v)))
        o_ref[...] = jnp.broadcast_to(-(jnp.sum(acc) / batch), (1, 1))

    return pl.pallas_call(
        body,
        out_shape=jax.ShapeDtypeStruct((1, 1), jnp.float32),
    )(x)


def kernel(nodeindex, v_i, v_j, negsamples, first_embeddings,
           second_embeddings, context_embeddings):
    del nodeindex, first_embeddings
    B = v_i.shape[0]
    K = negsamples.shape[0]
    L = 16
    NW = 32
    BW = B // NW

    dots = _sc_dots(v_i, v_j, negsamples.reshape(K * B),
                    second_embeddings, context_embeddings,
                    NW, BW, K, L)
    x = dots.reshape((NW * (1 + K) * BW * L) // 128, 128)
    block_rows = (BW * L) // 128
    loss = _tc_finalize(x, B, 1 + K, block_rows)
    return loss[0, 0]


# speedup vs baseline: 1.0684x; 1.0684x over previous
import functools

import jax
import jax.numpy as jnp
from jax import lax
from jax.experimental import pallas as pl
from jax.experimental.pallas import tpu as pltpu
from jax.experimental.pallas import tpu_sc as plsc


def _sc_dots(v_i, v_j, neg_flat, second, context, NW, BW, K, L):
    B = v_i.shape[0]
    D = second.shape[1]
    NC = NW // 16
    NCH = D // L
    NI = 2 + K
    ND = 1 + K

    mesh = plsc.VectorSubcoreMesh(core_axis_name="c", subcore_axis_name="s",
                                  num_cores=NC)

    @functools.partial(
        pl.kernel,
        mesh=mesh,
        out_type=jax.ShapeDtypeStruct((NW * ND, BW, L), jnp.float32),
        compiler_params=pltpu.CompilerParams(use_tc_tiling_on_sc=False),
        scratch_types=[
            pltpu.VMEM((NI, BW), jnp.int32),
            pltpu.VMEM((BW, D), jnp.float32),
            pltpu.VMEM((BW, D), jnp.float32),
            pltpu.VMEM((K, BW, D), jnp.float32),
            pltpu.VMEM((ND, BW, L), jnp.float32),
            pltpu.SemaphoreType.DMA,
            pltpu.SemaphoreType.DMA,
            pltpu.SemaphoreType.DMA,
        ],
    )
    def k(vi_hbm, vj_hbm, neg_hbm, second_hbm, context_hbm, out_hbm,
          idx_v, vi_rows, vj_rows, neg_rows, out_v, isem, sem, wsem):
        wid = lax.axis_index("s") * NC + lax.axis_index("c")
        base = wid * BW

        scps = [
            pltpu.async_copy(vi_hbm.at[pl.ds(base, BW)], idx_v.at[0], isem),
            pltpu.async_copy(vj_hbm.at[pl.ds(base, BW)], idx_v.at[1], isem),
        ]
        for kk in range(K):
            scps.append(
                pltpu.async_copy(neg_hbm.at[pl.ds(kk * B + base, BW)],
                                 idx_v.at[2 + kk], isem))
        for cp in scps:
            cp.wait()

        cps = [
            pltpu.async_copy(second_hbm.at[idx_v.at[0]], vi_rows, sem),
            pltpu.async_copy(context_hbm.at[idx_v.at[1]], vj_rows, sem),
        ]
        for kk in range(K):
            cps.append(
                pltpu.async_copy(context_hbm.at[idx_v.at[2 + kk]],
                                 neg_rows.at[kk], sem))
        cps[0].wait()
        cps[1].wait()

        @plsc.parallel_loop(0, BW, unroll=4)
        def pos_body(g):
            acc = vi_rows[g, pl.ds(0, L)] * vj_rows[g, pl.ds(0, L)]
            for c in range(1, NCH):
                acc = acc + (vi_rows[g, pl.ds(c * L, L)]
                             * vj_rows[g, pl.ds(c * L, L)])
            out_v[0, g, :] = acc

        for kk in range(K):
            cps[2 + kk].wait()

            @plsc.parallel_loop(0, BW, unroll=4)
            def neg_body(g, _kk=kk):
                acc = (vi_rows[g, pl.ds(0, L)]
                       * neg_rows[_kk, g, pl.ds(0, L)])
                for c in range(1, NCH):
                    acc = acc + (vi_rows[g, pl.ds(c * L, L)]
                                 * neg_rows[_kk, g, pl.ds(c * L, L)])
                out_v[1 + _kk, g, :] = acc

        pltpu.sync_copy(out_v, out_hbm.at[pl.ds(wid * ND, ND)])

    return k(v_i, v_j, neg_flat, second, context)


def _tc_finalize(x, batch, num_dots, block_rows):
    R, C = x.shape
    L = 16
    G = C // L

    def body(x_ref, o_ref):
        xs = x_ref[...]
        col = lax.broadcasted_iota(jnp.int32, (C, G), 0)
        grp = lax.broadcasted_iota(jnp.int32, (C, G), 1)
        a = (col // L == grp).astype(jnp.float32)
        s = jnp.dot(xs, a, preferred_element_type=jnp.float32)

        row = lax.broadcasted_iota(jnp.int32, (R, G), 0)
        d = (row // block_rows) % num_dots
        v = jnp.where(d == 0, s, -s)
        acc = jnp.minimum(v, 0.0) - jnp.log1p(jnp.exp(-jnp.abs(v)))
        o_ref[...] = jnp.broadcast_to(-(jnp.sum(acc) / batch), (1, 1))

    return pl.pallas_call(
        body,
        out_shape=jax.ShapeDtypeStruct((1, 1), jnp.float32),
    )(x)


def kernel(nodeindex, v_i, v_j, negsamples, first_embeddings,
           second_embeddings, context_embeddings):
    del nodeindex, first_embeddings
    B = v_i.shape[0]
    K = negsamples.shape[0]
    L = 16
    NW = 32
    BW = B // NW

    dots = _sc_dots(v_i, v_j, negsamples.reshape(K * B),
                    second_embeddings, context_embeddings,
                    NW, BW, K, L)
    x = dots.reshape((NW * (1 + K) * BW * L) // 128, 128)
    block_rows = (BW * L) // 128
    loss = _tc_finalize(x, B, 1 + K, block_rows)
    return loss[0, 0]
